# baseline (speedup 1.0000x reference)
import functools

import jax
import jax.numpy as jnp
from jax import lax
from jax.experimental import pallas as pl
from jax.experimental.pallas import tpu as pltpu
from jax.experimental.pallas import tpu_sc as plsc

_NC = 2
_NS = 16
_NW = _NC * _NS


@functools.partial(jax.jit, static_argnames=("b_per_w", "chunk"))
def _emb_lookup(x_flat, table, *, b_per_w, chunk):
    d = table.shape[1]
    b_total = x_flat.shape[0]
    n_chunks = b_per_w // chunk
    mesh = plsc.VectorSubcoreMesh(core_axis_name="c", subcore_axis_name="s")

    @functools.partial(
        pl.kernel,
        mesh=mesh,
        out_type=jax.ShapeDtypeStruct((b_total, d), jnp.float32),
        scratch_types=[
            pltpu.VMEM((b_per_w,), jnp.int32),
            pltpu.VMEM((chunk, d), jnp.float32),
            pltpu.VMEM((chunk, d), jnp.float32),
            pltpu.SemaphoreType.DMA,
            pltpu.SemaphoreType.DMA,
        ],
    )
    def body(idx_hbm, table_hbm, out_hbm, idx_v, rows_a, rows_b, g_sem, w_sem):
        wid = lax.axis_index("s") * _NC + lax.axis_index("c")
        base = wid * b_per_w
        bufs = (rows_a, rows_b)
        pltpu.sync_copy(idx_hbm.at[pl.ds(base, b_per_w)], idx_v)

        def gather(c):
            return pltpu.async_copy(
                table_hbm.at[idx_v.at[pl.ds(c * chunk, chunk)]],
                bufs[c % 2], g_sem)

        def writeback(c):
            return pltpu.async_copy(
                bufs[c % 2], out_hbm.at[pl.ds(base + c * chunk, chunk)], w_sem)

        gathers = [gather(0)]
        writes = []
        for c in range(n_chunks):
            gathers[c].wait()
            wb = writeback(c)
            writes.append(wb)
            if c + 1 < n_chunks:
                if c >= 1:
                    writes[c - 1].wait()
                gathers.append(gather(c + 1))
        writes[n_chunks - 2].wait()
        writes[n_chunks - 1].wait()

    return body(x_flat, table)


def kernel(x, table):
    b, s = x.shape
    x_flat = x.reshape(-1).astype(jnp.int32)
    out = _emb_lookup(x_flat, table, b_per_w=(b * s) // _NW, chunk=32)
    return out.reshape(b, s, table.shape[1])


# speedup vs baseline: 1.0613x; 1.0613x over previous
import functools

import jax
import jax.numpy as jnp
from jax import lax
from jax.experimental import pallas as pl
from jax.experimental.pallas import tpu as pltpu
from jax.experimental.pallas import tpu_sc as plsc

_NC = 2
_NS = 16
_NW = _NC * _NS


@functools.partial(jax.jit, static_argnames=("b_per_w", "chunk"))
def _emb_lookup(x_flat, table, *, b_per_w, chunk):
    d = table.shape[1]
    b_total = x_flat.shape[0]
    n_chunks = b_per_w // chunk
    mesh = plsc.VectorSubcoreMesh(core_axis_name="c", subcore_axis_name="s")

    @functools.partial(
        pl.kernel,
        mesh=mesh,
        out_type=jax.ShapeDtypeStruct((b_total, d), jnp.float32),
        scratch_types=[
            pltpu.VMEM((b_per_w,), jnp.int32),
            pltpu.VMEM((chunk, d), jnp.float32),
            pltpu.VMEM((chunk, d), jnp.float32),
            pltpu.VMEM((chunk, d), jnp.float32),
            pltpu.SemaphoreType.DMA,
            pltpu.SemaphoreType.DMA,
        ],
    )
    def body(idx_hbm, table_hbm, out_hbm, idx_v, rows_a, rows_b, rows_c,
             g_sem, w_sem):
        wid = lax.axis_index("s") * _NC + lax.axis_index("c")
        base = wid * b_per_w
        bufs = (rows_a, rows_b, rows_c)
        nbuf = len(bufs)
        pltpu.sync_copy(idx_hbm.at[pl.ds(base, b_per_w)], idx_v)

        def gather(c):
            return pltpu.async_copy(
                table_hbm.at[idx_v.at[pl.ds(c * chunk, chunk)]],
                bufs[c % nbuf], g_sem)

        def writeback(c):
            return pltpu.async_copy(
                bufs[c % nbuf], out_hbm.at[pl.ds(base + c * chunk, chunk)],
                w_sem)

        gathers = [gather(c) for c in range(min(nbuf - 1, n_chunks))]
        writes = []
        for c in range(n_chunks):
            gathers[c].wait()
            writes.append(writeback(c))
            nxt = c + nbuf - 1
            if nxt < n_chunks:
                if nxt >= nbuf:
                    writes[nxt - nbuf].wait()
                gathers.append(gather(nxt))
        for c in range(max(0, n_chunks - nbuf), n_chunks):
            writes[c].wait()

    return body(x_flat, table)


def kernel(x, table):
    b, s = x.shape
    x_flat = x.reshape(-1).astype(jnp.int32)
    out = _emb_lookup(x_flat, table, b_per_w=(b * s) // _NW, chunk=32)
    return out.reshape(b, s, table.shape[1])
